# Initial kernel scaffold; baseline (speedup 1.0000x reference)
import functools

import jax
import jax.numpy as jnp
from jax import lax
from jax.experimental import pallas as pl
from jax.experimental.pallas import tpu as pltpu
from jax.experimental.pallas import tpu_sc as plsc

N = 10000
E = 320000
D = 128
H = 256

NC = 2
NS = 16
NW = NC * NS
CHUNK = 128
CHUNKS_PER_W = 79
E_PER_W = CHUNK * CHUNKS_PER_W
E_PAD = E_PER_W * NW
ROWS_PER_S = 626
N_PAD = ROWS_PER_S * NS
TRASH = N + 8
EPS = 1e-5


def _sc_partial_segment_sum(x, src_w, dst_w, zeros):
    mesh = plsc.VectorSubcoreMesh(core_axis_name="c", subcore_axis_name="s")

    @functools.partial(
        pl.kernel,
        out_type=jax.ShapeDtypeStruct((NC * N_PAD, D), jnp.float32),
        mesh=mesh,
        scratch_types=[
            pltpu.VMEM((CHUNKS_PER_W, CHUNK), jnp.int32),
            pltpu.VMEM((CHUNKS_PER_W, CHUNK), jnp.int32),
            pltpu.VMEM((CHUNK, D), jnp.float32),
            pltpu.VMEM_SHARED((N_PAD, D), jnp.float32),
            pltpu.SemaphoreType.DMA,
        ],
    )
    def sc_kernel(x_hbm, src_hbm, dst_hbm, z_hbm, out_hbm,
                  src_v, dst_v, rows_v, acc, sem):
        c = lax.axis_index("c")
        s = lax.axis_index("s")
        wid = s * NC + c
        r0 = s * ROWS_PER_S
        pltpu.sync_copy(z_hbm.at[pl.ds(r0, ROWS_PER_S)],
                        acc.at[pl.ds(r0, ROWS_PER_S)])
        pltpu.sync_copy(src_hbm.at[wid], src_v)
        pltpu.sync_copy(dst_hbm.at[wid], dst_v)
        plsc.subcore_barrier()

        def body(j, carry):
            pltpu.async_copy(x_hbm.at[src_v.at[j]], rows_v, sem).wait()
            pltpu.sync_copy(rows_v, acc.at[dst_v.at[j]], add=True)
            return carry

        lax.fori_loop(0, CHUNKS_PER_W, body, 0)
        plsc.subcore_barrier()
        out_base = c * N_PAD + r0
        pltpu.sync_copy(acc.at[pl.ds(r0, ROWS_PER_S)],
                        out_hbm.at[pl.ds(out_base, ROWS_PER_S)])

    return sc_kernel(x, src_w, dst_w, zeros)


def _tc_dense(x, p0, p1, W_root, W_nbr, b_base, gamma1, beta1,
              W1, b1, W2, b2, gamma2, beta2):
    def body(x_ref, p0_ref, p1_ref, wr_ref, wn_ref, bb_ref, g1_ref, be1_ref,
             w1_ref, b1_ref, w2_ref, b2_ref, g2_ref, be2_ref, o_ref):
        xv = x_ref[...]
        agg = p0_ref[...] + p1_ref[...]
        h = jnp.dot(xv, wr_ref[...], preferred_element_type=jnp.float32)
        h = h + jnp.dot(agg, wn_ref[...], preferred_element_type=jnp.float32)
        h = h + bb_ref[...]
        mu = jnp.mean(h, axis=0, keepdims=True)
        hc = h - mu
        var = jnp.mean(hc * hc, axis=0, keepdims=True)
        h = hc * lax.rsqrt(var + EPS) * g1_ref[...] + be1_ref[...]
        h = jnp.maximum(h, 0.0) + xv
        t = jnp.maximum(
            jnp.dot(h, w1_ref[...], preferred_element_type=jnp.float32)
            + b1_ref[...], 0.0)
        y = (jnp.dot(t, w2_ref[...], preferred_element_type=jnp.float32)
             + b2_ref[...] + h)
        mu2 = jnp.mean(y, axis=0, keepdims=True)
        yc = y - mu2
        var2 = jnp.mean(yc * yc, axis=0, keepdims=True)
        o_ref[...] = yc * lax.rsqrt(var2 + EPS) * g2_ref[...] + be2_ref[...]

    return pl.pallas_call(
        body,
        out_shape=jax.ShapeDtypeStruct((N, D), jnp.float32),
    )(x, p0, p1, W_root, W_nbr,
      b_base.reshape(1, D), gamma1.reshape(1, D), beta1.reshape(1, D),
      W1, b1.reshape(1, H), W2, b2.reshape(1, D),
      gamma2.reshape(1, D), beta2.reshape(1, D))


def kernel(x, edge_index, W_root, W_nbr, b_base, gamma1, beta1,
           W1, b1, W2, b2, gamma2, beta2):
    src = edge_index[0]
    dst = edge_index[1]
    pad = E_PAD - E
    src_w = jnp.concatenate(
        [src, jnp.zeros((pad,), jnp.int32)]).reshape(NW, CHUNKS_PER_W, CHUNK)
    dst_w = jnp.concatenate(
        [dst, jnp.full((pad,), TRASH, jnp.int32)]).reshape(NW, CHUNKS_PER_W, CHUNK)
    zeros = jnp.zeros((N_PAD, D), jnp.float32)
    parts = _sc_partial_segment_sum(x, src_w, dst_w, zeros)
    p0 = parts[:N]
    p1 = parts[N_PAD:N_PAD + N]
    return _tc_dense(x, p0, p1, W_root, W_nbr, b_base, gamma1, beta1,
                     W1, b1, W2, b2, gamma2, beta2)


# speedup vs baseline: 5.0170x; 5.0170x over previous
import functools

import jax
import jax.numpy as jnp
from jax import lax
from jax.experimental import pallas as pl
from jax.experimental.pallas import tpu as pltpu
from jax.experimental.pallas import tpu_sc as plsc

N = 10000
E = 320000
D = 128
H = 256

NC = 2
NS = 16
NW = NC * NS
CHUNK = 128
CHUNKS_PER_W = 79
E_PER_W = CHUNK * CHUNKS_PER_W
E_PAD = E_PER_W * NW
ROWS_PER_S = 632
N_PAD = ROWS_PER_S * NS
TRASH = N + 8
EPS = 1e-5


def _sc_partial_segment_sum(x, src_w, dst_w, zeros):
    mesh = plsc.VectorSubcoreMesh(core_axis_name="c", subcore_axis_name="s")

    @functools.partial(
        pl.kernel,
        out_type=jax.ShapeDtypeStruct((NC * N_PAD, D), jnp.float32),
        mesh=mesh,
        scratch_types=[
            pltpu.VMEM((CHUNKS_PER_W, CHUNK), jnp.int32),
            pltpu.VMEM((CHUNKS_PER_W, CHUNK), jnp.int32),
            pltpu.VMEM((CHUNK, D), jnp.float32),
            pltpu.VMEM_SHARED((N_PAD, D), jnp.float32),
            pltpu.SemaphoreType.DMA,
        ],
    )
    def sc_kernel(x_hbm, src_hbm, dst_hbm, z_hbm, out_hbm,
                  src_v, dst_v, rows_v, acc, sem):
        c = lax.axis_index("c")
        s = lax.axis_index("s")
        wid = s * NC + c
        r0 = s * ROWS_PER_S
        pltpu.sync_copy(z_hbm.at[pl.ds(r0, ROWS_PER_S)],
                        acc.at[pl.ds(r0, ROWS_PER_S)])
        pltpu.sync_copy(src_hbm.at[wid], src_v)
        pltpu.sync_copy(dst_hbm.at[wid], dst_v)
        plsc.subcore_barrier()

        def body(j, carry):
            pltpu.async_copy(x_hbm.at[src_v.at[j]], rows_v, sem).wait()
            pltpu.sync_copy(rows_v, acc.at[dst_v.at[j]], add=True)
            return carry

        lax.fori_loop(0, CHUNKS_PER_W, body, 0)
        plsc.subcore_barrier()
        out_base = c * N_PAD + r0
        pltpu.sync_copy(acc.at[pl.ds(r0, ROWS_PER_S)],
                        out_hbm.at[pl.ds(out_base, ROWS_PER_S)])

    return sc_kernel(x, src_w, dst_w, zeros)


def _tc_dense(x, p0, p1, W_root, W_nbr, b_base, gamma1, beta1,
              W1, b1, W2, b2, gamma2, beta2):
    def body(x_ref, p0_ref, p1_ref, wr_ref, wn_ref, bb_ref, g1_ref, be1_ref,
             w1_ref, b1_ref, w2_ref, b2_ref, g2_ref, be2_ref, o_ref):
        xv = x_ref[...]
        agg = p0_ref[...] + p1_ref[...]
        h = jnp.dot(xv, wr_ref[...], preferred_element_type=jnp.float32)
        h = h + jnp.dot(agg, wn_ref[...], preferred_element_type=jnp.float32)
        h = h + bb_ref[...]
        mu = jnp.mean(h, axis=0, keepdims=True)
        hc = h - mu
        var = jnp.mean(hc * hc, axis=0, keepdims=True)
        h = hc * lax.rsqrt(var + EPS) * g1_ref[...] + be1_ref[...]
        h = jnp.maximum(h, 0.0) + xv
        t = jnp.maximum(
            jnp.dot(h, w1_ref[...], preferred_element_type=jnp.float32)
            + b1_ref[...], 0.0)
        y = (jnp.dot(t, w2_ref[...], preferred_element_type=jnp.float32)
             + b2_ref[...] + h)
        mu2 = jnp.mean(y, axis=0, keepdims=True)
        yc = y - mu2
        var2 = jnp.mean(yc * yc, axis=0, keepdims=True)
        o_ref[...] = yc * lax.rsqrt(var2 + EPS) * g2_ref[...] + be2_ref[...]

    return pl.pallas_call(
        body,
        out_shape=jax.ShapeDtypeStruct((N, D), jnp.float32),
    )(x, p0, p1, W_root, W_nbr,
      b_base.reshape(1, D), gamma1.reshape(1, D), beta1.reshape(1, D),
      W1, b1.reshape(1, H), W2, b2.reshape(1, D),
      gamma2.reshape(1, D), beta2.reshape(1, D))


def kernel(x, edge_index, W_root, W_nbr, b_base, gamma1, beta1,
           W1, b1, W2, b2, gamma2, beta2):
    src = edge_index[0]
    dst = edge_index[1]
    pad = E_PAD - E
    src_w = jnp.concatenate(
        [src, jnp.zeros((pad,), jnp.int32)]).reshape(NW, CHUNKS_PER_W, CHUNK)
    dst_w = jnp.concatenate(
        [dst, jnp.full((pad,), TRASH, jnp.int32)]).reshape(NW, CHUNKS_PER_W, CHUNK)
    zeros = jnp.zeros((N_PAD, D), jnp.float32)
    parts = _sc_partial_segment_sum(x, src_w, dst_w, zeros)
    p0 = parts[:N]
    p1 = parts[N_PAD:N_PAD + N]
    return _tc_dense(x, p0, p1, W_root, W_nbr, b_base, gamma1, beta1,
                     W1, b1, W2, b2, gamma2, beta2)
